# baseline (speedup 1.0000x reference)
import jax
import jax.numpy as jnp
from jax.experimental import pallas as pl

NUM_CODES = 1024
LATENT_DIM = 256
TOK_BLOCK = 2048
BATCHES_PER_BLOCK = 2


def _vq_body(zb_ref, w_ref, sw_ref, idx_ref, zq_ref):
    zb = zb_ref[...]
    w = w_ref[...]
    sz = jnp.sum(zb * zb, axis=1, keepdims=True)
    mm2 = jax.lax.dot_general(
        zb + zb, w, (((1,), (1,)), ((), ())),
        preferred_element_type=jnp.float32)
    d = (sz + sw_ref[...]) - mm2
    minv = jnp.min(d, axis=1, keepdims=True)
    iota = jax.lax.broadcasted_iota(jnp.int32, (TOK_BLOCK, NUM_CODES), 1)
    idx = jnp.min(jnp.where(d == minv, iota, NUM_CODES), axis=1)
    idx_ref[0] = idx[None, :]
    iota_t = jax.lax.broadcasted_iota(jnp.int32, (NUM_CODES, TOK_BLOCK), 0)
    onehot = (iota_t == idx[None, :]).astype(jnp.float32)
    zq = jax.lax.dot_general(
        w, onehot, (((0,), (0,)), ((), ())),
        preferred_element_type=jnp.float32)
    sp = TOK_BLOCK // BATCHES_PER_BLOCK
    for j in range(BATCHES_PER_BLOCK):
        zq_ref[j] = zq[:, j * sp:(j + 1) * sp]


def kernel(z, W):
    B, C, H, Wd = z.shape
    S = H * Wd
    ntok = B * S
    nblk = ntok // TOK_BLOCK
    zp = jnp.transpose(z, (0, 2, 3, 1)).reshape(ntok, LATENT_DIM)
    sw = jnp.sum(W ** 2, axis=1).reshape(1, NUM_CODES)

    idx3d, zqt = pl.pallas_call(
        _vq_body,
        grid=(nblk,),
        in_specs=[
            pl.BlockSpec((TOK_BLOCK, LATENT_DIM), lambda i: (i, 0)),
            pl.BlockSpec((NUM_CODES, LATENT_DIM), lambda i: (0, 0)),
            pl.BlockSpec((1, NUM_CODES), lambda i: (0, 0)),
        ],
        out_specs=[
            pl.BlockSpec((1, 1, TOK_BLOCK), lambda i: (i, 0, 0)),
            pl.BlockSpec((BATCHES_PER_BLOCK, LATENT_DIM, S),
                         lambda i: (i, 0, 0)),
        ],
        out_shape=[
            jax.ShapeDtypeStruct((nblk, 1, TOK_BLOCK), jnp.int32),
            jax.ShapeDtypeStruct((B, LATENT_DIM, S), jnp.float32),
        ],
    )(zp, W, sw)

    indices = idx3d.reshape(ntok)
    z_q = zqt.reshape(B, LATENT_DIM, H, Wd)
    return (indices, z_q)


# speedup vs baseline: 1.0049x; 1.0049x over previous
import jax
import jax.numpy as jnp
from jax.experimental import pallas as pl

NUM_CODES = 1024
LATENT_DIM = 256
TOK_BLOCK = 2048
BATCHES_PER_BLOCK = 2


def _vq_body(zb_ref, w_ref, sw_ref, idx_ref, zq_ref):
    zb = zb_ref[...]
    w = w_ref[...]
    sz = jnp.sum(zb * zb, axis=1, keepdims=True)
    mm = jax.lax.dot_general(
        zb, w, (((1,), (1,)), ((), ())),
        preferred_element_type=jnp.float32)
    d = (sz + sw_ref[...]) - 2.0 * mm
    minv = jnp.min(d, axis=1, keepdims=True)
    iota = jax.lax.broadcasted_iota(jnp.int32, (TOK_BLOCK, NUM_CODES), 1)
    idx = jnp.min(jnp.where(d == minv, iota, NUM_CODES), axis=1)
    idx_ref[0] = idx[None, :]
    iota_t = jax.lax.broadcasted_iota(jnp.int32, (NUM_CODES, TOK_BLOCK), 0)
    onehot = (iota_t == idx[None, :]).astype(jnp.float32)
    zq = jax.lax.dot_general(
        w, onehot, (((0,), (0,)), ((), ())),
        preferred_element_type=jnp.float32)
    sp = TOK_BLOCK // BATCHES_PER_BLOCK
    for j in range(BATCHES_PER_BLOCK):
        zq_ref[j] = zq[:, j * sp:(j + 1) * sp]


def kernel(z, W):
    B, C, H, Wd = z.shape
    S = H * Wd
    ntok = B * S
    nblk = ntok // TOK_BLOCK
    zp = jnp.transpose(z, (0, 2, 3, 1)).reshape(ntok, LATENT_DIM)
    sw = jnp.sum(W ** 2, axis=1).reshape(1, NUM_CODES)

    idx3d, zqt = pl.pallas_call(
        _vq_body,
        grid=(nblk,),
        in_specs=[
            pl.BlockSpec((TOK_BLOCK, LATENT_DIM), lambda i: (i, 0)),
            pl.BlockSpec((NUM_CODES, LATENT_DIM), lambda i: (0, 0)),
            pl.BlockSpec((1, NUM_CODES), lambda i: (0, 0)),
        ],
        out_specs=[
            pl.BlockSpec((1, 1, TOK_BLOCK), lambda i: (i, 0, 0)),
            pl.BlockSpec((BATCHES_PER_BLOCK, LATENT_DIM, S),
                         lambda i: (i, 0, 0)),
        ],
        out_shape=[
            jax.ShapeDtypeStruct((nblk, 1, TOK_BLOCK), jnp.int32),
            jax.ShapeDtypeStruct((B, LATENT_DIM, S), jnp.float32),
        ],
    )(zp, W, sw)

    indices = idx3d.reshape(ntok)
    z_q = zqt.reshape(B, LATENT_DIM, H, Wd)
    return (indices, z_q)
